# speedup vs baseline: 2.8956x; 2.8956x over previous
import jax
import jax.numpy as jnp
from jax.experimental import pallas as pl

N_PAD = 10240


def _matmul_body(x_ref, t_ref, o_ref):
    o_ref[...] = jnp.dot(x_ref[...], t_ref[...],
                         preferred_element_type=jnp.float32)


def _matmul(x, theta):
    blk = 1024
    return pl.pallas_call(
        _matmul_body,
        grid=(N_PAD // blk,),
        in_specs=[
            pl.BlockSpec((blk, 128), lambda i: (i, 0)),
            pl.BlockSpec((128, 128), lambda i: (0, 0)),
        ],
        out_specs=pl.BlockSpec((blk, 128), lambda i: (i, 0)),
        out_shape=jax.ShapeDtypeStruct((N_PAD, 128), jnp.float32),
    )(x, theta)


def kernel(data, edge_list, weight_list, theta):
    N = data.shape[0]
    src = jnp.concatenate([edge_list[:, 0], edge_list[:, 1]]).astype(jnp.int32)
    dst = jnp.concatenate([edge_list[:, 1], edge_list[:, 0]]).astype(jnp.int32)
    w = jnp.concatenate([weight_list, weight_list])
    deg = jax.ops.segment_sum(w, dst, num_segments=N) + 1.0
    d_inv = jax.lax.rsqrt(deg + 1e-10)
    xs = d_inv[:, None] * data
    acc = jax.ops.segment_sum(w[:, None] * xs[src], dst, num_segments=N)
    pre = d_inv[:, None] * (acc + xs)
    pre = jnp.pad(pre, ((0, N_PAD - N), (0, 0)))
    out = _matmul(pre, theta)
    return out[:N]


# speedup vs baseline: 14.3027x; 4.9394x over previous
import jax
import jax.numpy as jnp
from jax import lax
from jax.experimental import pallas as pl
from jax.experimental.pallas import tpu as pltpu
from jax.experimental.pallas import tpu_sc as plsc

NP = 10240
D = 128
NW = 32
NCHUNK = 160
EPT = NCHUNK * 128
E2P = NW * EPT
IDX_ROWS = E2P // 128


def _deg_body(dst_hbm, w_hbm, out_hbm, dst_v, w_v, deg_s, zb):
    c = lax.axis_index("c")
    s = lax.axis_index("s")
    wid = s * 2 + c
    for q in range(640 // 16):
        zb[pl.ds(q * 16, 16)] = jnp.zeros((16,), jnp.float32)
    pltpu.sync_copy(zb, deg_s.at[pl.ds(s * 640, 640)])
    plsc.subcore_barrier()
    pltpu.sync_copy(dst_hbm.at[pl.ds(wid * NCHUNK, NCHUNK)], dst_v)
    pltpu.sync_copy(w_hbm.at[pl.ds(wid * NCHUNK, NCHUNK)], w_v)

    def body(j, carry):
        pltpu.sync_copy(w_v.at[j], deg_s.at[dst_v.at[j]], add=True)
        return carry

    lax.fori_loop(0, NCHUNK, body, 0)
    plsc.subcore_barrier()
    pltpu.sync_copy(deg_s.at[pl.ds(s * 640, 640)],
                    out_hbm.at[pl.ds(c * NP + s * 640, 640)])


def _degrees(dst2d, w2d):
    return pl.kernel(
        _deg_body,
        out_type=jax.ShapeDtypeStruct((2 * NP,), jnp.float32),
        mesh=plsc.VectorSubcoreMesh(core_axis_name="c", subcore_axis_name="s", num_cores=2, num_subcores=16),
        scratch_types=[
            pltpu.VMEM((NCHUNK, 128), jnp.int32),
            pltpu.VMEM((NCHUNK, 128), jnp.float32),
            pltpu.VMEM_SHARED((NP,), jnp.float32),
            pltpu.VMEM((640,), jnp.float32),
        ],
    )(dst2d, w2d)


NGRP = NCHUNK // 8


def _mp_body(src_hbm, dst_hbm, w_hbm, xs_hbm, out_hbm,
             src_v, dst_v, w_v, rows_v, acc_s, gsem, isem):
    c = lax.axis_index("c")
    s = lax.axis_index("s")
    wid = s * 2 + c

    def zbody(i, carry):
        for q in range(8):
            rows_v[i, pl.ds(q * 16, 16)] = jnp.zeros((16,), jnp.float32)
        return carry

    lax.fori_loop(0, 128, zbody, 0)
    for r in range(5):
        pltpu.sync_copy(rows_v.at[pl.ds(0, 128)],
                        acc_s.at[pl.ds(s * 640 + r * 128, 128)])
    plsc.subcore_barrier()

    base = wid * NCHUNK

    def start_idx_group(g, half):
        pltpu.async_copy(src_hbm.at[pl.ds(base + g * 8, 8)],
                         src_v.at[pl.ds(half, 8)], isem)
        pltpu.async_copy(dst_hbm.at[pl.ds(base + g * 8, 8)],
                         dst_v.at[pl.ds(half, 8)], isem)
        pltpu.async_copy(w_hbm.at[pl.ds(base + g * 8, 8)],
                         w_v.at[pl.ds(half, 8)], isem)

    def wait_idx_group(half):
        pltpu.make_async_copy(src_hbm.at[pl.ds(0, 8)],
                              src_v.at[pl.ds(half, 8)], isem).wait()
        pltpu.make_async_copy(dst_hbm.at[pl.ds(0, 8)],
                              dst_v.at[pl.ds(half, 8)], isem).wait()
        pltpu.make_async_copy(w_hbm.at[pl.ds(0, 8)],
                              w_v.at[pl.ds(half, 8)], isem).wait()

    start_idx_group(0, 0)
    wait_idx_group(0)
    start_idx_group(1, 8)
    pltpu.async_copy(xs_hbm.at[src_v.at[0]], rows_v.at[pl.ds(0, 128)], gsem)

    def loop(j, carry):
        b = lax.rem(j, 2) * 128
        jdiv = lax.div(j, 8)
        sub = lax.rem(j, 8)
        sel = lax.rem(jdiv, 2) * 8
        pltpu.make_async_copy(xs_hbm.at[src_v.at[sel + sub]],
                              rows_v.at[pl.ds(b, 128)], gsem).wait()

        @pl.when(jnp.logical_and(sub == 7, j + 1 < NCHUNK))
        def _():
            wait_idx_group(8 - sel)

        @pl.when(j + 1 < NCHUNK)
        def _():
            j1 = j + 1
            r1 = lax.rem(lax.div(j1, 8), 2) * 8 + lax.rem(j1, 8)
            pltpu.async_copy(xs_hbm.at[src_v.at[r1]],
                             rows_v.at[pl.ds(128 - b, 128)], gsem)

        def sbody(g, carry2):
            w16 = w_v[sel + sub, pl.ds(g * 16, 16)]
            for t in range(16):
                wb = w16.at[jnp.full((16,), t, jnp.int32)].get(
                    mode="promise_in_bounds", unique_indices=False)
                row = b + g * 16 + t
                for q in range(8):
                    sl = pl.ds(q * 16, 16)
                    rows_v[row, sl] = rows_v[row, sl] * wb
            return carry2

        lax.fori_loop(0, 8, sbody, 0)
        pltpu.sync_copy(rows_v.at[pl.ds(b, 128)],
                        acc_s.at[dst_v.at[sel + sub]], add=True)

        @pl.when(jnp.logical_and(sub == 7, jdiv + 2 < NGRP))
        def _():
            start_idx_group(jdiv + 2, sel)

        return carry

    lax.fori_loop(0, NCHUNK, loop, 0)
    plsc.subcore_barrier()
    pltpu.sync_copy(acc_s.at[pl.ds(s * 640, 640)],
                    out_hbm.at[pl.ds(c * NP + s * 640, 640)])


def _message_pass(src2d, dst2d, w2d, xs):
    return pl.kernel(
        _mp_body,
        out_type=jax.ShapeDtypeStruct((2 * NP, D), jnp.float32),
        mesh=plsc.VectorSubcoreMesh(core_axis_name="c", subcore_axis_name="s", num_cores=2, num_subcores=16),
        scratch_types=[
            pltpu.VMEM((16, 128), jnp.int32),
            pltpu.VMEM((16, 128), jnp.int32),
            pltpu.VMEM((16, 128), jnp.float32),
            pltpu.VMEM((256, D), jnp.float32),
            pltpu.VMEM_SHARED((NP, D), jnp.float32),
            pltpu.SemaphoreType.DMA,
            pltpu.SemaphoreType.DMA,
        ],
    )(src2d, dst2d, w2d, xs)


def _dinv_body(dg_ref, o_ref):
    o_ref[...] = lax.rsqrt(dg_ref[0] + dg_ref[1] + (1.0 + 1e-10))


def _xs_body(x_ref, di_ref, o_ref):
    o_ref[...] = x_ref[...] * di_ref[...]


def _out_body(a0_ref, a1_ref, xs_ref, di_ref, th_ref, o_ref):
    pre = (a0_ref[...] + a1_ref[...] + xs_ref[...]) * di_ref[...]
    o_ref[...] = jnp.dot(pre, th_ref[...], preferred_element_type=jnp.float32)


def kernel(data, edge_list, weight_list, theta):
    n = data.shape[0]
    e0 = edge_list[:, 0].astype(jnp.int32)
    e1 = edge_list[:, 1].astype(jnp.int32)
    e2 = 2 * edge_list.shape[0]
    pad = E2P - e2
    src = jnp.concatenate([e0, e1, jnp.zeros((pad,), jnp.int32)])
    dst = jnp.concatenate([e1, e0, jnp.zeros((pad,), jnp.int32)])
    w2 = jnp.concatenate([weight_list, weight_list,
                          jnp.zeros((pad,), jnp.float32)])
    src2d = src.reshape(IDX_ROWS, 128)
    dst2d = dst.reshape(IDX_ROWS, 128)
    w2d = w2.reshape(IDX_ROWS, 128)
    datap = jnp.pad(data, ((0, NP - n), (0, 0)))

    deg_parts = _degrees(dst2d, w2d)

    dinv2d = pl.pallas_call(
        _dinv_body,
        out_shape=jax.ShapeDtypeStruct((NP // 128, 128), jnp.float32),
    )(deg_parts.reshape(2, NP // 128, 128))
    dinv_col = dinv2d.reshape(NP, 1)

    blk = 1024
    grid = NP // blk
    xs = pl.pallas_call(
        _xs_body,
        grid=(grid,),
        in_specs=[
            pl.BlockSpec((blk, D), lambda i: (i, 0)),
            pl.BlockSpec((blk, 1), lambda i: (i, 0)),
        ],
        out_specs=pl.BlockSpec((blk, D), lambda i: (i, 0)),
        out_shape=jax.ShapeDtypeStruct((NP, D), jnp.float32),
    )(datap, dinv_col)

    acc_parts = _message_pass(src2d, dst2d, w2d, xs)

    out = pl.pallas_call(
        _out_body,
        grid=(grid,),
        in_specs=[
            pl.BlockSpec((blk, D), lambda i: (i, 0)),
            pl.BlockSpec((blk, D), lambda i: (i, 0)),
            pl.BlockSpec((blk, D), lambda i: (i, 0)),
            pl.BlockSpec((blk, 1), lambda i: (i, 0)),
            pl.BlockSpec((D, D), lambda i: (0, 0)),
        ],
        out_specs=pl.BlockSpec((blk, D), lambda i: (i, 0)),
        out_shape=jax.ShapeDtypeStruct((NP, D), jnp.float32),
    )(acc_parts[:NP], acc_parts[NP:], xs, dinv_col, theta)
    return out[:n]
